# baseline (speedup 1.0000x reference)
import functools
import jax
import jax.numpy as jnp
from jax.experimental import pallas as pl
from jax.experimental.pallas import tpu as pltpu

_RADIUS = 0.1
_NCPAD = 128


def _gelu(v):
    return 0.5 * v * (1.0 + jax.lax.erf(v * 0.7071067811865476))


def _conv_body(xT_ref, fT_ref, cmin_ref, cmax_ref, xbT_ref, fb_ref, WaTt_ref,
               Wab_ref, ba_ref, Wb_ref, bb_ref, out_ref, *, n_chunks, bj):
    r2 = _RADIUS * _RADIUS
    Q = jnp.dot(fb_ref[...], Wab_ref[...],
                preferred_element_type=jnp.float32) + ba_ref[...]
    xbT = xbT_ref[...]
    bi = xbT.shape[1]
    w = Q.shape[1]

    def gap(d):
        lo = jnp.min(xbT[d, :])
        hi = jnp.max(xbT[d, :])
        return jnp.maximum(
            jnp.maximum(cmin_ref[d:d + 1, :] - hi, lo - cmax_ref[d:d + 1, :]),
            0.0)

    gx, gy, gz = gap(0), gap(1), gap(2)
    keepf = ((gx * gx + gy * gy + gz * gz) < r2).astype(jnp.float32)
    lane = jax.lax.broadcasted_iota(jnp.int32, (1, _NCPAD), 1)

    def chunk_work(c, S, cnt):
        xj = xT_ref[:, pl.ds(c * bj, bj)]
        fj = fT_ref[:, pl.ds(c * bj, bj)]
        PjT = jnp.dot(WaTt_ref[...], fj,
                      preferred_element_type=jnp.float32)
        diff = xbT[:, :, None] - xj[:, None, :]
        d2 = jnp.sum(diff * diff, axis=0)
        m = d2 < r2
        H = _gelu(Q[:, :, None] + PjT[None, :, :])
        Hm = jnp.where(m[:, None, :], H, 0.0)
        S = S + jnp.sum(Hm, axis=2)
        cnt = cnt + jnp.sum(m.astype(jnp.float32), axis=1, keepdims=True)
        return S, cnt

    def body(c, carry):
        S, cnt = carry
        pred = jnp.sum(jnp.where(lane == c, keepf, 0.0)) > 0.5
        return jax.lax.cond(pred, lambda s, n: chunk_work(c, s, n),
                            lambda s, n: (s, n), S, cnt)

    S0 = jnp.zeros((bi, w), jnp.float32)
    c0 = jnp.zeros((bi, 1), jnp.float32)
    S, cnt = jax.lax.fori_loop(0, n_chunks, body, (S0, c0))
    mean = S / jnp.maximum(cnt, 1.0)
    out_ref[...] = jnp.dot(mean, Wb_ref[...],
                           preferred_element_type=jnp.float32) + bb_ref[...]


def _conv(x, feat, cmin, cmax, Wa, ba, Wb, bb, bi=128, bj=128):
    n = x.shape[0]
    cf = feat.shape[1]
    w = Wa.shape[1]
    nc = n // bj
    xT = x.T
    fT = feat.T
    WaTt = Wa[:cf].T
    Wab = Wa[cf:]
    ba2 = ba.reshape(1, w)
    bb2 = bb.reshape(1, w)
    grid = n // bi
    body = functools.partial(_conv_body, n_chunks=nc, bj=bj)
    return pl.pallas_call(
        body,
        grid=(grid,),
        in_specs=[
            pl.BlockSpec((3, n), lambda i: (0, 0)),
            pl.BlockSpec((cf, n), lambda i: (0, 0)),
            pl.BlockSpec((3, _NCPAD), lambda i: (0, 0)),
            pl.BlockSpec((3, _NCPAD), lambda i: (0, 0)),
            pl.BlockSpec((3, bi), lambda i: (0, i)),
            pl.BlockSpec((bi, cf), lambda i: (i, 0)),
            pl.BlockSpec((w, cf), lambda i: (0, 0)),
            pl.BlockSpec((cf, w), lambda i: (0, 0)),
            pl.BlockSpec((1, w), lambda i: (0, 0)),
            pl.BlockSpec((w, w), lambda i: (0, 0)),
            pl.BlockSpec((1, w), lambda i: (0, 0)),
        ],
        out_specs=pl.BlockSpec((bi, w), lambda i: (i, 0)),
        out_shape=jax.ShapeDtypeStruct((n, w), jnp.float32),
    )(xT, fT, cmin, cmax, xT, feat, WaTt, Wab, ba2, Wb, bb2)


def _head_body(h_ref, wT_ref, b_ref, out_ref):
    g = _gelu(h_ref[...])
    out_ref[...] = (jnp.sum(g * wT_ref[...], axis=1, keepdims=True)
                    + b_ref[...])


def _head(h, Wout, bout):
    n, w = h.shape
    return pl.pallas_call(
        _head_body,
        in_specs=[
            pl.BlockSpec((n, w), lambda: (0, 0)),
            pl.BlockSpec((1, w), lambda: (0, 0)),
            pl.BlockSpec((1, 1), lambda: (0, 0)),
        ],
        out_specs=pl.BlockSpec((n, 1), lambda: (0, 0)),
        out_shape=jax.ShapeDtypeStruct((n, 1), jnp.float32),
    )(h, Wout.T, bout.reshape(1, 1))


def kernel(x_in, W1a, b1a, W1b, b1b, W2a, b2a, W2b, b2b, Wout, bout):
    x = jnp.squeeze(x_in)
    n = x.shape[0]
    bj = 128
    key = 2.0 * jnp.floor(x[:, 2] * (1.0 / _RADIUS)) + x[:, 1]
    perm = jnp.argsort(key)
    xs = x[perm]
    npad = -n % bj
    xp = jnp.concatenate(
        [xs, jnp.full((npad, x.shape[1]), 100.0, jnp.float32)], axis=0)
    nc = xp.shape[0] // bj
    cb = xp.reshape(nc, bj, 3)
    pad = jnp.full((3, _NCPAD - nc), 1e6, jnp.float32)
    cmin = jnp.concatenate([cb.min(axis=1).T, pad], axis=1)
    cmax = jnp.concatenate([cb.max(axis=1).T, -pad], axis=1)
    h1 = _conv(xp, xp, cmin, cmax, W1a, b1a, W1b, b1b)
    h2 = _conv(xp, h1, cmin, cmax, W2a, b2a, W2b, b2b)
    res = _head(h2[:n], Wout, bout)
    return jnp.zeros((n, 1), jnp.float32).at[perm].set(res)


# speedup vs baseline: 4.0106x; 4.0106x over previous
import functools
import jax
import jax.numpy as jnp
from jax.experimental import pallas as pl
from jax.experimental.pallas import tpu as pltpu

_RADIUS = 0.1


def _gelu(v):
    return 0.5 * v * (1.0 + jax.lax.erf(v * 0.7071067811865476))


def _conv_body(visit_ref, cnt_ref, xT_ref, fT_ref, xbT_ref, fb_ref, WaTt_ref,
               Wab_ref, ba_ref, Wb_ref, bb_ref, out_ref, *, bj):
    r2 = _RADIUS * _RADIUS
    i = pl.program_id(0)
    Q = jnp.dot(fb_ref[...], Wab_ref[...],
                preferred_element_type=jnp.float32) + ba_ref[...]
    xbT = xbT_ref[...]
    bi = xbT.shape[1]
    w = Q.shape[1]

    def body(k, carry):
        S, cnt = carry
        c = visit_ref[i, k]
        xj = xT_ref[:, pl.ds(c * bj, bj)]
        fj = fT_ref[:, pl.ds(c * bj, bj)]
        PjT = jnp.dot(WaTt_ref[...], fj,
                      preferred_element_type=jnp.float32)
        diff = xbT[:, :, None] - xj[:, None, :]
        d2 = jnp.sum(diff * diff, axis=0)
        m = d2 < r2
        H = _gelu(Q[:, :, None] + PjT[None, :, :])
        Hm = jnp.where(m[:, None, :], H, 0.0)
        S = S + jnp.sum(Hm, axis=2)
        cnt = cnt + jnp.sum(m.astype(jnp.float32), axis=1, keepdims=True)
        return S, cnt

    S0 = jnp.zeros((bi, w), jnp.float32)
    c0 = jnp.zeros((bi, 1), jnp.float32)
    S, cnt = jax.lax.fori_loop(0, cnt_ref[i], body, (S0, c0))
    mean = S / jnp.maximum(cnt, 1.0)
    out_ref[...] = jnp.dot(mean, Wb_ref[...],
                           preferred_element_type=jnp.float32) + bb_ref[...]


def _conv(x, feat, visit, vcnt, Wa, ba, Wb, bb, bi, bj):
    n = x.shape[0]
    cf = feat.shape[1]
    w = Wa.shape[1]
    xT = x.T
    fT = feat.T
    WaTt = Wa[:cf].T
    Wab = Wa[cf:]
    ba2 = ba.reshape(1, w)
    bb2 = bb.reshape(1, w)
    grid = n // bi
    body = functools.partial(_conv_body, bj=bj)
    return pl.pallas_call(
        body,
        grid_spec=pltpu.PrefetchScalarGridSpec(
            num_scalar_prefetch=2,
            grid=(grid,),
            in_specs=[
                pl.BlockSpec((3, n), lambda i, *_: (0, 0)),
                pl.BlockSpec((cf, n), lambda i, *_: (0, 0)),
                pl.BlockSpec((3, bi), lambda i, *_: (0, i)),
                pl.BlockSpec((bi, cf), lambda i, *_: (i, 0)),
                pl.BlockSpec((w, cf), lambda i, *_: (0, 0)),
                pl.BlockSpec((cf, w), lambda i, *_: (0, 0)),
                pl.BlockSpec((1, w), lambda i, *_: (0, 0)),
                pl.BlockSpec((w, w), lambda i, *_: (0, 0)),
                pl.BlockSpec((1, w), lambda i, *_: (0, 0)),
            ],
            out_specs=pl.BlockSpec((bi, w), lambda i, *_: (i, 0)),
        ),
        out_shape=jax.ShapeDtypeStruct((n, w), jnp.float32),
    )(visit, vcnt, xT, fT, xT, feat, WaTt, Wab, ba2, Wb, bb2)


def _head_body(h_ref, wT_ref, b_ref, out_ref):
    g = _gelu(h_ref[...])
    out_ref[...] = (jnp.sum(g * wT_ref[...], axis=1, keepdims=True)
                    + b_ref[...])


def _head(h, Wout, bout):
    n, w = h.shape
    return pl.pallas_call(
        _head_body,
        in_specs=[
            pl.BlockSpec((n, w), lambda: (0, 0)),
            pl.BlockSpec((1, w), lambda: (0, 0)),
            pl.BlockSpec((1, 1), lambda: (0, 0)),
        ],
        out_specs=pl.BlockSpec((n, 1), lambda: (0, 0)),
        out_shape=jax.ShapeDtypeStruct((n, 1), jnp.float32),
    )(h, Wout.T, bout.reshape(1, 1))


def kernel(x_in, W1a, b1a, W1b, b1b, W2a, b2a, W2b, b2b, Wout, bout):
    x = jnp.squeeze(x_in)
    n = x.shape[0]
    bi = 128
    bj = 128
    key = 2.0 * jnp.floor(x[:, 2] * (1.0 / _RADIUS)) + x[:, 1]
    perm = jnp.argsort(key)
    xs = x[perm]
    npad = -n % max(bi, bj)
    xp = jnp.concatenate(
        [xs, jnp.full((npad, x.shape[1]), 100.0, jnp.float32)], axis=0)
    npts = xp.shape[0]
    nc = npts // bj
    grid = npts // bi
    cbb = xp.reshape(nc, bj, 3)
    bbb = xp.reshape(grid, bi, 3)
    cmin, cmax = cbb.min(axis=1), cbb.max(axis=1)
    bmin, bmax = bbb.min(axis=1), bbb.max(axis=1)
    gaps = jnp.maximum(
        jnp.maximum(cmin[None, :, :] - bmax[:, None, :],
                    bmin[:, None, :] - cmax[None, :, :]), 0.0)
    keep = jnp.sum(gaps * gaps, axis=2) < _RADIUS * _RADIUS
    visit = jnp.argsort(jnp.where(keep, 0, 1), axis=1,
                        stable=True).astype(jnp.int32)
    vcnt = jnp.sum(keep, axis=1).astype(jnp.int32)
    h1 = _conv(xp, xp, visit, vcnt, W1a, b1a, W1b, b1b, bi, bj)
    h2 = _conv(xp, h1, visit, vcnt, W2a, b2a, W2b, b2b, bi, bj)
    res = _head(h2[:n], Wout, bout)
    return jnp.zeros((n, 1), jnp.float32).at[perm].set(res)


# speedup vs baseline: 5.1590x; 1.2863x over previous
import functools
import jax
import jax.numpy as jnp
from jax.experimental import pallas as pl
from jax.experimental.pallas import tpu as pltpu

_RADIUS = 0.1


def _gelu(v):
    return 0.5 * v * (1.0 + jax.lax.erf(v * 0.7071067811865476))


def _pq_body(fT_ref, WaTt_ref, out_ref):
    out_ref[...] = jnp.dot(WaTt_ref[...], fT_ref[...],
                           preferred_element_type=jnp.float32)


def _conv_body(visit_ref, cnt_ref, xT_ref, PT_ref, xbT_ref, fb_ref,
               Wab_ref, ba_ref, Wb_ref, bb_ref, out_ref, *, bj):
    r2 = _RADIUS * _RADIUS
    i = pl.program_id(0)
    Q = jnp.dot(fb_ref[...], Wab_ref[...],
                preferred_element_type=jnp.float32) + ba_ref[...]
    xbT = xbT_ref[...]
    bi = xbT.shape[1]
    w = Q.shape[1]
    ones_j = jnp.ones((bj,), jnp.float32)

    def body(k, carry):
        S, cnt = carry
        c = visit_ref[i, k]
        xj = xT_ref[:, pl.ds(c * bj, bj)]
        PjT = PT_ref[:, pl.ds(c * bj, bj)]
        diff = xbT[:, :, None] - xj[:, None, :]
        d2 = jnp.sum(diff * diff, axis=0)
        m = d2 < r2
        H = _gelu(Q[:, :, None] + PjT[None, :, :])
        Hm = jnp.where(m[:, None, :], H, 0.0)
        S = S + jnp.dot(Hm, ones_j,
                        preferred_element_type=jnp.float32)
        cnt = cnt + jnp.dot(m.astype(jnp.float32), ones_j,
                            preferred_element_type=jnp.float32)
        return S, cnt

    S0 = jnp.zeros((bi, w), jnp.float32)
    c0 = jnp.zeros((bi,), jnp.float32)
    S, cnt = jax.lax.fori_loop(0, cnt_ref[i], body, (S0, c0))
    mean = S / jnp.maximum(cnt, 1.0)[:, None]
    out_ref[...] = jnp.dot(mean, Wb_ref[...],
                           preferred_element_type=jnp.float32) + bb_ref[...]


def _conv(x, feat, visit, vcnt, Wa, ba, Wb, bb, bi, bj):
    n = x.shape[0]
    cf = feat.shape[1]
    w = Wa.shape[1]
    xT = x.T
    fT = feat.T
    WaTt = Wa[:cf].T
    Wab = Wa[cf:]
    ba2 = ba.reshape(1, w)
    bb2 = bb.reshape(1, w)
    grid = n // bi
    PT = pl.pallas_call(
        _pq_body,
        in_specs=[
            pl.BlockSpec((cf, n), lambda: (0, 0)),
            pl.BlockSpec((w, cf), lambda: (0, 0)),
        ],
        out_specs=pl.BlockSpec((w, n), lambda: (0, 0)),
        out_shape=jax.ShapeDtypeStruct((w, n), jnp.float32),
    )(fT, WaTt)
    body = functools.partial(_conv_body, bj=bj)
    return pl.pallas_call(
        body,
        grid_spec=pltpu.PrefetchScalarGridSpec(
            num_scalar_prefetch=2,
            grid=(grid,),
            in_specs=[
                pl.BlockSpec((3, n), lambda i, *_: (0, 0)),
                pl.BlockSpec((w, n), lambda i, *_: (0, 0)),
                pl.BlockSpec((3, bi), lambda i, *_: (0, i)),
                pl.BlockSpec((bi, cf), lambda i, *_: (i, 0)),
                pl.BlockSpec((cf, w), lambda i, *_: (0, 0)),
                pl.BlockSpec((1, w), lambda i, *_: (0, 0)),
                pl.BlockSpec((w, w), lambda i, *_: (0, 0)),
                pl.BlockSpec((1, w), lambda i, *_: (0, 0)),
            ],
            out_specs=pl.BlockSpec((bi, w), lambda i, *_: (i, 0)),
        ),
        out_shape=jax.ShapeDtypeStruct((n, w), jnp.float32),
    )(visit, vcnt, xT, PT, xT, feat, Wab, ba2, Wb, bb2)


def _head_body(h_ref, wT_ref, b_ref, out_ref):
    g = _gelu(h_ref[...])
    out_ref[...] = (jnp.sum(g * wT_ref[...], axis=1, keepdims=True)
                    + b_ref[...])


def _head(h, Wout, bout):
    n, w = h.shape
    return pl.pallas_call(
        _head_body,
        in_specs=[
            pl.BlockSpec((n, w), lambda: (0, 0)),
            pl.BlockSpec((1, w), lambda: (0, 0)),
            pl.BlockSpec((1, 1), lambda: (0, 0)),
        ],
        out_specs=pl.BlockSpec((n, 1), lambda: (0, 0)),
        out_shape=jax.ShapeDtypeStruct((n, 1), jnp.float32),
    )(h, Wout.T, bout.reshape(1, 1))


def kernel(x_in, W1a, b1a, W1b, b1b, W2a, b2a, W2b, b2b, Wout, bout):
    x = jnp.squeeze(x_in)
    n = x.shape[0]
    bi = 256
    bj = 256
    perm = jnp.argsort(x[:, 2])
    xs = x[perm]
    npad = -n % max(bi, bj)
    xp = jnp.concatenate(
        [xs, jnp.full((npad, x.shape[1]), 100.0, jnp.float32)], axis=0)
    npts = xp.shape[0]
    nc = npts // bj
    grid = npts // bi
    cbb = xp.reshape(nc, bj, 3)
    bbb = xp.reshape(grid, bi, 3)
    cmin, cmax = cbb.min(axis=1), cbb.max(axis=1)
    bmin, bmax = bbb.min(axis=1), bbb.max(axis=1)
    gaps = jnp.maximum(
        jnp.maximum(cmin[None, :, :] - bmax[:, None, :],
                    bmin[:, None, :] - cmax[None, :, :]), 0.0)
    keep = jnp.sum(gaps * gaps, axis=2) < _RADIUS * _RADIUS
    visit = jnp.argsort(jnp.where(keep, 0, 1), axis=1,
                        stable=True).astype(jnp.int32)
    vcnt = jnp.sum(keep, axis=1).astype(jnp.int32)
    h1 = _conv(xp, xp, visit, vcnt, W1a, b1a, W1b, b1b, bi, bj)
    h2 = _conv(xp, h1, visit, vcnt, W2a, b2a, W2b, b2b, bi, bj)
    res = _head(h2[:n], Wout, bout)
    return jnp.zeros((n, 1), jnp.float32).at[perm].set(res)


# speedup vs baseline: 5.2015x; 1.0082x over previous
import functools
import jax
import jax.numpy as jnp
from jax.experimental import pallas as pl
from jax.experimental.pallas import tpu as pltpu

_RADIUS = 0.1


def _gelu(v):
    return 0.5 * v * (1.0 + jax.lax.erf(v * 0.7071067811865476))


def _conv_body(xT_ref, fT_ref, zlo_ref, zhi_ref, xbT_ref, fb_ref, WaTt_ref,
               Wab_ref, ba_ref, Wb_ref, bb_ref, out_ref, *, n_chunks, bj):
    r2 = _RADIUS * _RADIUS
    Q = jnp.dot(fb_ref[...], Wab_ref[...],
                preferred_element_type=jnp.float32) + ba_ref[...]
    xbT = xbT_ref[...]
    bi = xbT.shape[1]
    w = Q.shape[1]

    zb = xbT[2, :]
    zb_lo = jnp.min(zb)
    zb_hi = jnp.max(zb)
    c_start = jnp.sum((zhi_ref[...] < zb_lo - _RADIUS).astype(jnp.int32))
    c_end = n_chunks - jnp.sum(
        (zlo_ref[...] > zb_hi + _RADIUS).astype(jnp.int32))

    def chunk(c, carry):
        S, cnt = carry
        xj = xT_ref[:, pl.ds(c * bj, bj)]
        fj = fT_ref[:, pl.ds(c * bj, bj)]
        PjT = jnp.dot(WaTt_ref[...], fj,
                      preferred_element_type=jnp.float32)
        diff = xbT[:, :, None] - xj[:, None, :]
        d2 = jnp.sum(diff * diff, axis=0)
        m = d2 < r2
        H = _gelu(Q[:, :, None] + PjT[None, :, :])
        Hm = jnp.where(m[:, None, :], H, 0.0)
        S = S + jnp.sum(Hm, axis=2)
        cnt = cnt + jnp.sum(m.astype(jnp.float32), axis=1, keepdims=True)
        return S, cnt

    S0 = jnp.zeros((bi, w), jnp.float32)
    c0 = jnp.zeros((bi, 1), jnp.float32)
    S, cnt = jax.lax.fori_loop(c_start, c_end, chunk, (S0, c0))
    mean = S / jnp.maximum(cnt, 1.0)
    out_ref[...] = jnp.dot(mean, Wb_ref[...],
                           preferred_element_type=jnp.float32) + bb_ref[...]


def _conv(x, feat, zlo, zhi, Wa, ba, Wb, bb, bi=256, bj=256):
    n = x.shape[0]
    cf = feat.shape[1]
    w = Wa.shape[1]
    nc = n // bj
    xT = x.T
    fT = feat.T
    WaTt = Wa[:cf].T
    Wab = Wa[cf:]
    ba2 = ba.reshape(1, w)
    bb2 = bb.reshape(1, w)
    grid = n // bi
    body = functools.partial(_conv_body, n_chunks=nc, bj=bj)
    return pl.pallas_call(
        body,
        grid=(grid,),
        in_specs=[
            pl.BlockSpec((3, n), lambda i: (0, 0)),
            pl.BlockSpec((cf, n), lambda i: (0, 0)),
            pl.BlockSpec((1, nc), lambda i: (0, 0)),
            pl.BlockSpec((1, nc), lambda i: (0, 0)),
            pl.BlockSpec((3, bi), lambda i: (0, i)),
            pl.BlockSpec((bi, cf), lambda i: (i, 0)),
            pl.BlockSpec((w, cf), lambda i: (0, 0)),
            pl.BlockSpec((cf, w), lambda i: (0, 0)),
            pl.BlockSpec((1, w), lambda i: (0, 0)),
            pl.BlockSpec((w, w), lambda i: (0, 0)),
            pl.BlockSpec((1, w), lambda i: (0, 0)),
        ],
        out_specs=pl.BlockSpec((bi, w), lambda i: (i, 0)),
        out_shape=jax.ShapeDtypeStruct((n, w), jnp.float32),
    )(xT, fT, zlo, zhi, xT, feat, WaTt, Wab, ba2, Wb, bb2)


def _head_body(h_ref, wT_ref, b_ref, out_ref):
    g = _gelu(h_ref[...])
    out_ref[...] = (jnp.sum(g * wT_ref[...], axis=1, keepdims=True)
                    + b_ref[...])


def _head(h, Wout, bout):
    n, w = h.shape
    return pl.pallas_call(
        _head_body,
        in_specs=[
            pl.BlockSpec((n, w), lambda: (0, 0)),
            pl.BlockSpec((1, w), lambda: (0, 0)),
            pl.BlockSpec((1, 1), lambda: (0, 0)),
        ],
        out_specs=pl.BlockSpec((n, 1), lambda: (0, 0)),
        out_shape=jax.ShapeDtypeStruct((n, 1), jnp.float32),
    )(h, Wout.T, bout.reshape(1, 1))


def kernel(x_in, W1a, b1a, W1b, b1b, W2a, b2a, W2b, b2b, Wout, bout):
    x = jnp.squeeze(x_in)
    n = x.shape[0]
    bj = 256
    perm = jnp.argsort(x[:, 2])
    xs = x[perm]
    npad = -n % 256
    xp = jnp.concatenate(
        [xs, jnp.full((npad, x.shape[1]), 100.0, jnp.float32)], axis=0)
    zcol = xp[:, 2]
    zlo = zcol[0::bj].reshape(1, -1)
    zhi = zcol[bj - 1::bj].reshape(1, -1)
    h1 = _conv(xp, xp, zlo, zhi, W1a, b1a, W1b, b1b)
    h2 = _conv(xp, h1, zlo, zhi, W2a, b2a, W2b, b2b)
    res = _head(h2[:n], Wout, bout)
    return jnp.zeros((n, 1), jnp.float32).at[perm].set(res)
